# baseline (speedup 1.0000x reference)
import jax
import jax.numpy as jnp
from jax.experimental import pallas as pl

B = 2
T = 2048
HIDDEN = 2048
HEADS = 16
D = HIDDEN // HEADS
H3 = 3 * HIDDEN


def _qkv_body(h_ref, w_ref, b_ref, o_ref):
    acc = jax.lax.dot_general(
        h_ref[...], w_ref[...],
        (((1,), (1,)), ((), ())),
        preferred_element_type=jnp.float32,
    )
    o_ref[...] = (acc + b_ref[...]).astype(jnp.bfloat16)


def _qkv_proj(hidden_bf, Wcat_bf, bcat):
    RM, CN = 512, 2048
    grid = (H3 // CN, (B * T) // RM)
    return pl.pallas_call(
        _qkv_body,
        grid=grid,
        in_specs=[
            pl.BlockSpec((RM, HIDDEN), lambda c, r: (r, 0)),
            pl.BlockSpec((CN, HIDDEN), lambda c, r: (c, 0)),
            pl.BlockSpec((1, CN), lambda c, r: (0, c)),
        ],
        out_specs=pl.BlockSpec((RM, CN), lambda c, r: (r, c)),
        out_shape=jax.ShapeDtypeStruct((B * T, H3), jnp.bfloat16),
    )(hidden_bf, Wcat_bf, bcat)


def _z_body(q_ref, k_ref, wp_ref, qp_ref, kpT_ref):
    yq = q_ref[0]
    yk = k_ref[0]
    wp = wp_ref[...].astype(jnp.bfloat16)
    accq = jnp.zeros((D, HEADS), jnp.float32)
    acck = jnp.zeros((HEADS, D), jnp.float32)
    for a in range(16):
        wpa = wp[:, a * D:(a + 1) * D]
        accq = accq + jax.lax.dot_general(
            yq[:, a, :], wpa, (((1,), (1,)), ((), ())),
            preferred_element_type=jnp.float32)
        acck = acck + jax.lax.dot_general(
            wpa, yk[:, a, :], (((1,), (1,)), ((), ())),
            preferred_element_type=jnp.float32)
    qp_ref[0] = accq
    kpT_ref[0] = acck


def _z_proj(qkv4, Wp):
    return pl.pallas_call(
        _z_body,
        grid=(B, HEADS),
        in_specs=[
            pl.BlockSpec((1, D, 16, D), lambda b, c: (b, 0, 0, c)),
            pl.BlockSpec((1, D, 16, D), lambda b, c: (b, 0, 0, c + HEADS)),
            pl.BlockSpec((HEADS, HIDDEN), lambda b, c: (0, 0)),
        ],
        out_specs=[
            pl.BlockSpec((1, D, HEADS), lambda b, c: (b, c, 0)),
            pl.BlockSpec((1, HEADS, D), lambda b, c: (b, 0, c)),
        ],
        out_shape=[
            jax.ShapeDtypeStruct((B, T, HEADS), jnp.float32),
            jax.ShapeDtypeStruct((B, HEADS, T), jnp.float32),
        ],
    )(qkv4, qkv4, Wp)


def _attn_body(qp_ref, kpT_ref, v_ref, lb_ref, o_ref):
    h = pl.program_id(1)
    lane = jax.lax.broadcasted_iota(jnp.int32, (1, HEADS), 1)
    hmask = (lane == h).astype(jnp.float32)
    lb = jnp.sum(lb_ref[...] * hmask)
    inv_bw = 1.0 / (jax.nn.softplus(lb) + 1e-4)
    zq = jnp.sum(qp_ref[0] * hmask, axis=1, keepdims=True)
    sub = jax.lax.broadcasted_iota(jnp.int32, (HEADS, 1), 0)
    smask = (sub == h).astype(jnp.float32)
    zk = jnp.sum(kpT_ref[0] * smask, axis=0, keepdims=True)
    zqs = zq * inv_bw
    zks = zk * inv_bw
    a = jnp.maximum(1.0 - jnp.abs(zqs - zks), 0.0)
    ctx = jax.lax.dot_general(
        a.astype(jnp.bfloat16), v_ref[0],
        (((1,), (0,)), ((), ())),
        preferred_element_type=jnp.float32,
    )
    o_ref[0] = ctx * (1.0 / T)


def _attention(q_proj, k_projT, v3d, lb2d):
    TQ = 512
    return pl.pallas_call(
        _attn_body,
        grid=(B, HEADS, T // TQ),
        in_specs=[
            pl.BlockSpec((1, TQ, HEADS), lambda b, h, t: (b, t, 0)),
            pl.BlockSpec((1, HEADS, T), lambda b, h, t: (b, 0, 0)),
            pl.BlockSpec((1, T, D), lambda b, h, t: (b, 0, 2 * HEADS + h)),
            pl.BlockSpec((1, HEADS), lambda b, h, t: (0, 0)),
        ],
        out_specs=pl.BlockSpec((1, TQ, D), lambda b, h, t: (b, t, h)),
        out_shape=jax.ShapeDtypeStruct((B, T, HIDDEN), jnp.float32),
    )(q_proj, k_projT, v3d, lb2d)


def kernel(hidden_states, Wq, bq, Wk, bk, Wv, bv, Wp, log_bandwidth):
    hidden_bf = hidden_states.reshape(B * T, HIDDEN).astype(jnp.bfloat16)
    Wcat_bf = jnp.concatenate([Wq, Wk, Wv], axis=0).astype(jnp.bfloat16)
    bcat = jnp.concatenate([bq, bk, bv]).reshape(1, H3)

    qkv = _qkv_proj(hidden_bf, Wcat_bf, bcat)

    qkv4 = qkv.reshape(B, D, 16, H3)
    q_proj, k_projT = _z_proj(qkv4, Wp)

    v3d = qkv.reshape(B, T, H3)
    lb2d = log_bandwidth.reshape(1, HEADS)
    return _attention(q_proj, k_projT, v3d, lb2d)


# speedup vs baseline: 1.2359x; 1.2359x over previous
import jax
import jax.numpy as jnp
from jax.experimental import pallas as pl

B = 2
T = 2048
HIDDEN = 2048
HEADS = 16
D = HIDDEN // HEADS
H3 = 3 * HIDDEN


def _qkv_body(h_ref, wq_ref, wk_ref, wv_ref, bq_ref, bk_ref, bv_ref,
              q_ref, k_ref, v_ref):
    ht = h_ref[...].astype(jnp.bfloat16)
    for w_ref, b_ref, o_ref in ((wq_ref, bq_ref, q_ref),
                                (wk_ref, bk_ref, k_ref),
                                (wv_ref, bv_ref, v_ref)):
        acc = jax.lax.dot_general(
            ht, w_ref[...],
            (((1,), (1,)), ((), ())),
            preferred_element_type=jnp.float32,
        )
        o_ref[...] = acc + b_ref[...]


def _qkv_proj(hidden2d, Wq_bf, Wk_bf, Wv_bf, bq2, bk2, bv2):
    RM = 512
    grid = ((B * T) // RM,)
    w_spec = pl.BlockSpec((HIDDEN, HIDDEN), lambda r: (0, 0))
    b_spec = pl.BlockSpec((1, HIDDEN), lambda r: (0, 0))
    o_spec = pl.BlockSpec((RM, HIDDEN), lambda r: (r, 0))
    o_shape = jax.ShapeDtypeStruct((B * T, HIDDEN), jnp.float32)
    return pl.pallas_call(
        _qkv_body,
        grid=grid,
        in_specs=[
            pl.BlockSpec((RM, HIDDEN), lambda r: (r, 0)),
            w_spec, w_spec, w_spec, b_spec, b_spec, b_spec,
        ],
        out_specs=[o_spec, o_spec, o_spec],
        out_shape=[o_shape, o_shape, o_shape],
    )(hidden2d, Wq_bf, Wk_bf, Wv_bf, bq2, bk2, bv2)


def _z_body(q_ref, k_ref, wp_ref, qp_ref, kpT_ref):
    yq = q_ref[0]
    yk = k_ref[0]
    wp = wp_ref[...]
    accq = jnp.zeros((D, HEADS), jnp.float32)
    acck = jnp.zeros((HEADS, D), jnp.float32)
    for a in range(16):
        wpa = wp[:, a * D:(a + 1) * D]
        accq = accq + jax.lax.dot_general(
            yq[:, a, :], wpa, (((1,), (1,)), ((), ())),
            preferred_element_type=jnp.float32)
        acck = acck + jax.lax.dot_general(
            wpa, yk[:, a, :], (((1,), (1,)), ((), ())),
            preferred_element_type=jnp.float32)
    qp_ref[0] = accq
    kpT_ref[0] = acck


def _z_proj(q4, k4, Wp):
    in_spec = pl.BlockSpec((1, D, 16, D), lambda b, c: (b, 0, 0, c))
    return pl.pallas_call(
        _z_body,
        grid=(B, HEADS),
        in_specs=[
            in_spec, in_spec,
            pl.BlockSpec((HEADS, HIDDEN), lambda b, c: (0, 0)),
        ],
        out_specs=[
            pl.BlockSpec((1, D, HEADS), lambda b, c: (b, c, 0)),
            pl.BlockSpec((1, HEADS, D), lambda b, c: (b, 0, c)),
        ],
        out_shape=[
            jax.ShapeDtypeStruct((B, T, HEADS), jnp.float32),
            jax.ShapeDtypeStruct((B, HEADS, T), jnp.float32),
        ],
    )(q4, k4, Wp)


def _attn_body(qp_ref, kpT_ref, v_ref, lb_ref, o_ref):
    h = pl.program_id(1)
    lane = jax.lax.broadcasted_iota(jnp.int32, (1, HEADS), 1)
    hmask = (lane == h).astype(jnp.float32)
    lb = jnp.sum(lb_ref[...] * hmask)
    inv_bw = 1.0 / (jax.nn.softplus(lb) + 1e-4)
    zq = jnp.sum(qp_ref[0] * hmask, axis=1, keepdims=True)
    sub = jax.lax.broadcasted_iota(jnp.int32, (HEADS, 1), 0)
    smask = (sub == h).astype(jnp.float32)
    zk = jnp.sum(kpT_ref[0] * smask, axis=0, keepdims=True)
    zqs = zq * inv_bw
    zks = zk * inv_bw
    a = jnp.maximum(1.0 - jnp.abs(zqs - zks), 0.0)
    ctx = jax.lax.dot_general(
        a, v_ref[0],
        (((1,), (0,)), ((), ())),
        preferred_element_type=jnp.float32,
    )
    o_ref[0] = ctx * (1.0 / T)


def _attention(q_proj, k_projT, v3d, lb2d):
    TQ = 1024
    return pl.pallas_call(
        _attn_body,
        grid=(B, HEADS, T // TQ),
        in_specs=[
            pl.BlockSpec((1, TQ, HEADS), lambda b, h, t: (b, t, 0)),
            pl.BlockSpec((1, HEADS, T), lambda b, h, t: (b, 0, 0)),
            pl.BlockSpec((1, T, D), lambda b, h, t: (b, 0, h)),
            pl.BlockSpec((1, HEADS), lambda b, h, t: (0, 0)),
        ],
        out_specs=pl.BlockSpec((1, TQ, D), lambda b, h, t: (b, t, h)),
        out_shape=jax.ShapeDtypeStruct((B, T, HIDDEN), jnp.float32),
    )(q_proj, k_projT, v3d, lb2d)


def kernel(hidden_states, Wq, bq, Wk, bk, Wv, bv, Wp, log_bandwidth):
    hidden2d = hidden_states.reshape(B * T, HIDDEN)
    qfull, kfull, vfull = _qkv_proj(
        hidden2d,
        Wq.astype(jnp.bfloat16), Wk.astype(jnp.bfloat16),
        Wv.astype(jnp.bfloat16),
        bq.reshape(1, HIDDEN), bk.reshape(1, HIDDEN), bv.reshape(1, HIDDEN))

    q4 = qfull.reshape(B, D, 16, HIDDEN)
    k4 = kfull.reshape(B, D, 16, HIDDEN)
    q_proj, k_projT = _z_proj(q4, k4, Wp)

    lb2d = log_bandwidth.reshape(1, HEADS)
    return _attention(q_proj, k_projT, vfull.reshape(B, T, HIDDEN), lb2d)
